# baseline (speedup 1.0000x reference)
import functools

import jax
import jax.numpy as jnp
from jax import lax
from jax.experimental import pallas as pl
from jax.experimental.pallas import tpu as pltpu
from jax.experimental.pallas import tpu_sc as plsc

D_MODEL = 1024
MEMORY_SIZE = 100000
NUM_HEADS = 4

_NC = 2
_NS = 16
_NW = _NC * _NS

_CHUNK = 32
_HALF = D_MODEL // 2
_L = 16


def _round_word(ab, bb):
    ra = ab + (0x7FFF + (lax.shift_right_logical(ab, 16) & 1))
    rb = bb + (0x7FFF + (lax.shift_right_logical(bb, 16) & 1))
    return lax.shift_right_logical(ra, 16) | (rb & jnp.int32(-65536))


def _make_sc_gather(n_rows):
    rows_per_w = n_rows // _NW
    n_chunks = rows_per_w // _CHUNK

    def _gather_body(table_hbm, idx_hbm, out_hbm, idx_v, rows0, rows1, obuf,
                     gs0, gs1):
        bufs, gsems = (rows0, rows1), (gs0, gs1)
        wid = lax.axis_index("s") * _NC + lax.axis_index("c")
        base = wid * rows_per_w
        pltpu.sync_copy(idx_hbm.at[pl.ds(base, rows_per_w)], idx_v)

        def _start_gather(j, b):
            return pltpu.async_copy(
                table_hbm.at[idx_v.at[pl.ds(j * _CHUNK, _CHUNK)]],
                bufs[b], gsems[b])

        def _wait_gather(j, b):
            pltpu.make_async_copy(
                table_hbm.at[idx_v.at[pl.ds(j * _CHUNK, _CHUNK)]],
                bufs[b], gsems[b]).wait()

        def _convert_and_flush(i, b):
            buf = bufs[b]

            @pl.loop(0, _CHUNK)
            def _row(r):
                for j in range(_HALF // _L):
                    a = buf[r, pl.ds(j * _L, _L)]
                    c = buf[r, pl.ds(_HALF + j * _L, _L)]
                    obuf[r, pl.ds(j * _L, _L)] = _round_word(a, c)

            pltpu.sync_copy(obuf, out_hbm.at[pl.ds(base + i * _CHUNK, _CHUNK)])

        _start_gather(0, 0)
        _start_gather(1, 1)

        @pl.loop(0, n_chunks - 2, step=2)
        def _main(g0):
            for b in range(2):
                i = g0 + b
                _wait_gather(i, b)
                _start_gather(i + 2, b)
                _convert_and_flush(i, b)
        for i in (n_chunks - 2, n_chunks - 1):
            b = i % 2
            _wait_gather(i, b)
            _convert_and_flush(i, b)

    return functools.partial(
        pl.kernel,
        out_type=jax.ShapeDtypeStruct((n_rows, _HALF), jnp.int32),
        mesh=plsc.VectorSubcoreMesh(core_axis_name="c", subcore_axis_name="s"),
        scratch_types=[
            pltpu.VMEM((rows_per_w,), jnp.int32),
            pltpu.VMEM((_CHUNK, D_MODEL), jnp.int32),
            pltpu.VMEM((_CHUNK, D_MODEL), jnp.int32),
            pltpu.VMEM((_CHUNK, _HALF), jnp.int32),
            pltpu.SemaphoreType.DMA,
            pltpu.SemaphoreType.DMA,
        ],
    )(_gather_body)


_sc_gather = _make_sc_gather(NUM_HEADS * 4 * 2048)


_BT = 512


def _mm_body(g_ref, mh0, mh1, mh2, mh3, w_ref, b_ref, h_ref, o_ref):
    g = g_ref[0]
    acc = None
    for hd, mh in enumerate((mh0, mh1, mh2, mh3)):
        w = mh[...]
        lo = lax.bitcast_convert_type(
            lax.shift_left(w, 16), jnp.float32).astype(jnp.bfloat16)
        hi = lax.bitcast_convert_type(
            w & jnp.int32(-65536), jnp.float32).astype(jnp.bfloat16)
        for half, x in ((0, lo), (1, hi)):
            c0 = hd * D_MODEL + half * _HALF
            part = lax.dot_general(
                x, w_ref[:, c0:c0 + _HALF],
                (((1,), (1,)), ((), ())),
                preferred_element_type=jnp.float32,
            )
            acc = part if acc is None else acc + part
    o_ref[...] = (1.0 - g) * h_ref[...] + g * (acc + b_ref[...])


def _tc_project(multi, w16, b2, hidden2d, gate):
    t = hidden2d.shape[0]
    nblk = t // _BT
    mh_specs = [
        pl.BlockSpec((_BT, _HALF), lambda i, hd=hd: (hd * nblk + i, 0))
        for hd in range(NUM_HEADS)
    ]
    return pl.pallas_call(
        _mm_body,
        grid=(nblk,),
        in_specs=[
            pl.BlockSpec(memory_space=pltpu.SMEM),
            *mh_specs,
            pl.BlockSpec((D_MODEL, NUM_HEADS * D_MODEL), lambda i: (0, 0)),
            pl.BlockSpec((1, D_MODEL), lambda i: (0, 0)),
            pl.BlockSpec((_BT, D_MODEL), lambda i: (i, 0)),
        ],
        out_specs=pl.BlockSpec((_BT, D_MODEL), lambda i: (i, 0)),
        out_shape=jax.ShapeDtypeStruct((t, D_MODEL), jnp.float32),
        compiler_params=pltpu.CompilerParams(
            dimension_semantics=("arbitrary",),
        ),
    )(gate, multi, multi, multi, multi, w16, b2, hidden2d)


def kernel(hidden_states, input_ids, memory_table, hash_coeffs, W, b, gate):
    bsz, seq, d = hidden_states.shape
    h = hash_coeffs.shape[0]
    t = bsz * seq

    ids_f = input_ids.reshape(-1)[None, :].astype(jnp.float32)
    idx = ((ids_f * hash_coeffs[:, None]) % MEMORY_SIZE).astype(jnp.int32)
    flat_idx = idx.reshape(-1)

    w16 = W.astype(jnp.bfloat16)
    b2 = b.reshape(1, d)
    hidden2d = hidden_states.reshape(t, d)

    table_bits = lax.bitcast_convert_type(memory_table, jnp.int32)
    multi = _sc_gather(table_bits, flat_idx)
    out = _tc_project(multi, w16, b2, hidden2d, gate)
    return out.reshape(bsz, seq, d)


# speedup vs baseline: 1.3027x; 1.3027x over previous
import functools

import jax
import jax.numpy as jnp
from jax import lax
from jax.experimental import pallas as pl
from jax.experimental.pallas import tpu as pltpu
from jax.experimental.pallas import tpu_sc as plsc

D_MODEL = 1024
MEMORY_SIZE = 100000
NUM_HEADS = 4

_NC = 2
_NS = 16
_NW = _NC * _NS

_CHUNK = 32
_HALF = D_MODEL // 2
_L = 16


def _round_word(ab, bb):
    ra = ab + (0x7FFF + (lax.shift_right_logical(ab, 16) & 1))
    rb = bb + (0x7FFF + (lax.shift_right_logical(bb, 16) & 1))
    return lax.shift_right_logical(ra, 16) | (rb & jnp.int32(-65536))


def _make_sc_gather(n_rows):
    rows_per_w = n_rows // _NW
    n_chunks = rows_per_w // _CHUNK

    def _gather_body(table_hbm, idx_hbm, out_hbm, idx_v, rows0, rows1, obuf,
                     gs0, gs1):
        bufs, gsems = (rows0, rows1), (gs0, gs1)
        wid = lax.axis_index("s") * _NC + lax.axis_index("c")
        base = wid * rows_per_w
        pltpu.sync_copy(idx_hbm.at[pl.ds(base, rows_per_w)], idx_v)

        def _start_gather(j, b):
            return pltpu.async_copy(
                table_hbm.at[idx_v.at[pl.ds(j * _CHUNK, _CHUNK)]],
                bufs[b], gsems[b])

        def _wait_gather(j, b):
            pltpu.make_async_copy(
                table_hbm.at[idx_v.at[pl.ds(j * _CHUNK, _CHUNK)]],
                bufs[b], gsems[b]).wait()

        def _convert_and_flush(i, b):
            buf = bufs[b]

            @plsc.parallel_loop(0, _CHUNK, unroll=4)
            def _row(r):
                for j in range(_HALF // _L):
                    a = buf[r, pl.ds(j * _L, _L)]
                    c = buf[r, pl.ds(_HALF + j * _L, _L)]
                    obuf[r, pl.ds(j * _L, _L)] = _round_word(a, c)

            pltpu.sync_copy(obuf, out_hbm.at[pl.ds(base + i * _CHUNK, _CHUNK)])

        _start_gather(0, 0)
        _start_gather(1, 1)

        @pl.loop(0, n_chunks - 2, step=2)
        def _main(g0):
            for b in range(2):
                i = g0 + b
                _wait_gather(i, b)
                _start_gather(i + 2, b)
                _convert_and_flush(i, b)
        for i in (n_chunks - 2, n_chunks - 1):
            b = i % 2
            _wait_gather(i, b)
            _convert_and_flush(i, b)

    return functools.partial(
        pl.kernel,
        out_type=jax.ShapeDtypeStruct((n_rows, _HALF), jnp.int32),
        mesh=plsc.VectorSubcoreMesh(core_axis_name="c", subcore_axis_name="s"),
        scratch_types=[
            pltpu.VMEM((rows_per_w,), jnp.int32),
            pltpu.VMEM((_CHUNK, D_MODEL), jnp.int32),
            pltpu.VMEM((_CHUNK, D_MODEL), jnp.int32),
            pltpu.VMEM((_CHUNK, _HALF), jnp.int32),
            pltpu.SemaphoreType.DMA,
            pltpu.SemaphoreType.DMA,
        ],
    )(_gather_body)


_sc_gather = _make_sc_gather(NUM_HEADS * 4 * 2048)


_BT = 512


def _mm_body(g_ref, mh0, mh1, mh2, mh3, w_ref, b_ref, h_ref, o_ref):
    g = g_ref[0]
    acc = None
    for hd, mh in enumerate((mh0, mh1, mh2, mh3)):
        w = mh[...]
        lo = lax.bitcast_convert_type(
            lax.shift_left(w, 16), jnp.float32).astype(jnp.bfloat16)
        hi = lax.bitcast_convert_type(
            w & jnp.int32(-65536), jnp.float32).astype(jnp.bfloat16)
        for half, x in ((0, lo), (1, hi)):
            c0 = hd * D_MODEL + half * _HALF
            part = lax.dot_general(
                x, w_ref[:, c0:c0 + _HALF],
                (((1,), (1,)), ((), ())),
                preferred_element_type=jnp.float32,
            )
            acc = part if acc is None else acc + part
    o_ref[...] = (1.0 - g) * h_ref[...] + g * (acc + b_ref[...])


def _tc_project(multi, w16, b2, hidden2d, gate):
    t = hidden2d.shape[0]
    nblk = t // _BT
    mh_specs = [
        pl.BlockSpec((_BT, _HALF), lambda i, hd=hd: (hd * nblk + i, 0))
        for hd in range(NUM_HEADS)
    ]
    return pl.pallas_call(
        _mm_body,
        grid=(nblk,),
        in_specs=[
            pl.BlockSpec(memory_space=pltpu.SMEM),
            *mh_specs,
            pl.BlockSpec((D_MODEL, NUM_HEADS * D_MODEL), lambda i: (0, 0)),
            pl.BlockSpec((1, D_MODEL), lambda i: (0, 0)),
            pl.BlockSpec((_BT, D_MODEL), lambda i: (i, 0)),
        ],
        out_specs=pl.BlockSpec((_BT, D_MODEL), lambda i: (i, 0)),
        out_shape=jax.ShapeDtypeStruct((t, D_MODEL), jnp.float32),
        compiler_params=pltpu.CompilerParams(
            dimension_semantics=("arbitrary",),
        ),
    )(gate, multi, multi, multi, multi, w16, b2, hidden2d)


def kernel(hidden_states, input_ids, memory_table, hash_coeffs, W, b, gate):
    bsz, seq, d = hidden_states.shape
    h = hash_coeffs.shape[0]
    t = bsz * seq

    ids_f = input_ids.reshape(-1)[None, :].astype(jnp.float32)
    idx = ((ids_f * hash_coeffs[:, None]) % MEMORY_SIZE).astype(jnp.int32)
    flat_idx = idx.reshape(-1)

    w16 = W.astype(jnp.bfloat16)
    b2 = b.reshape(1, d)
    hidden2d = hidden_states.reshape(t, d)

    table_bits = lax.bitcast_convert_type(memory_table, jnp.int32)
    multi = _sc_gather(table_bits, flat_idx)
    out = _tc_project(multi, w16, b2, hidden2d, gate)
    return out.reshape(bsz, seq, d)


# speedup vs baseline: 3.0204x; 2.3186x over previous
import functools

import jax
import jax.numpy as jnp
from jax import lax
from jax.experimental import pallas as pl
from jax.experimental.pallas import tpu as pltpu
from jax.experimental.pallas import tpu_sc as plsc

D_MODEL = 1024
MEMORY_SIZE = 100000
NUM_HEADS = 4

_NC = 2
_NS = 16
_NW = _NC * _NS

_CHUNK = 32
_NBUF = 3


def _make_sc_gather(n_rows):
    rows_per_w = n_rows // _NW
    n_chunks = rows_per_w // _CHUNK
    depth = _NBUF - 1

    def _gather_body(table_hbm, idx_hbm, out_hbm, idx_v, *scr):
        bufs = scr[:_NBUF]
        gsems = scr[_NBUF:2 * _NBUF]
        osems = scr[2 * _NBUF:]
        wid = lax.axis_index("s") * _NC + lax.axis_index("c")
        base = wid * rows_per_w
        pltpu.sync_copy(idx_hbm.at[pl.ds(base, rows_per_w)], idx_v)

        def _start_gather(j):
            b = j % _NBUF
            return pltpu.async_copy(
                table_hbm.at[idx_v.at[pl.ds(j * _CHUNK, _CHUNK)]],
                bufs[b], gsems[b])

        gathers = [None] * _NBUF
        out_copies = [None] * _NBUF
        for j in range(min(depth, n_chunks)):
            gathers[j % _NBUF] = _start_gather(j)
        for i in range(n_chunks):
            b = i % _NBUF
            gathers[b].wait()
            gathers[b] = None
            out_copies[b] = pltpu.async_copy(
                bufs[b], out_hbm.at[pl.ds(base + i * _CHUNK, _CHUNK)],
                osems[b])
            j = i + depth
            if j < n_chunks:
                bj = j % _NBUF
                if out_copies[bj] is not None:
                    out_copies[bj].wait()
                    out_copies[bj] = None
                gathers[bj] = _start_gather(j)
        for oc in out_copies:
            if oc is not None:
                oc.wait()

    return functools.partial(
        pl.kernel,
        out_type=jax.ShapeDtypeStruct((n_rows, D_MODEL), jnp.float32),
        mesh=plsc.VectorSubcoreMesh(core_axis_name="c", subcore_axis_name="s"),
        scratch_types=[
            pltpu.VMEM((rows_per_w,), jnp.int32),
        ] + [pltpu.VMEM((_CHUNK, D_MODEL), jnp.float32)] * _NBUF
          + [pltpu.SemaphoreType.DMA] * (2 * _NBUF),
    )(_gather_body)


_sc_gather = _make_sc_gather(NUM_HEADS * 4 * 2048)


_BT = 512


def _mm_body(g_ref, mh0, mh1, mh2, mh3, w_ref, b_ref, h_ref, o_ref):
    g = g_ref[0]
    acc = None
    for hd, mh in enumerate((mh0, mh1, mh2, mh3)):
        part = lax.dot_general(
            mh[...].astype(jnp.bfloat16),
            w_ref[:, hd * D_MODEL:(hd + 1) * D_MODEL],
            (((1,), (1,)), ((), ())),
            preferred_element_type=jnp.float32,
        )
        acc = part if acc is None else acc + part
    o_ref[...] = (1.0 - g) * h_ref[...] + g * (acc + b_ref[...])


def _tc_project(multi, w16, b2, hidden2d, gate):
    t = hidden2d.shape[0]
    nblk = t // _BT
    mh_specs = [
        pl.BlockSpec((_BT, D_MODEL), lambda i, hd=hd: (hd * nblk + i, 0))
        for hd in range(NUM_HEADS)
    ]
    return pl.pallas_call(
        _mm_body,
        grid=(nblk,),
        in_specs=[
            pl.BlockSpec(memory_space=pltpu.SMEM),
            *mh_specs,
            pl.BlockSpec((D_MODEL, NUM_HEADS * D_MODEL), lambda i: (0, 0)),
            pl.BlockSpec((1, D_MODEL), lambda i: (0, 0)),
            pl.BlockSpec((_BT, D_MODEL), lambda i: (i, 0)),
        ],
        out_specs=pl.BlockSpec((_BT, D_MODEL), lambda i: (i, 0)),
        out_shape=jax.ShapeDtypeStruct((t, D_MODEL), jnp.float32),
        compiler_params=pltpu.CompilerParams(
            dimension_semantics=("arbitrary",),
        ),
    )(gate, multi, multi, multi, multi, w16, b2, hidden2d)


def kernel(hidden_states, input_ids, memory_table, hash_coeffs, W, b, gate):
    bsz, seq, d = hidden_states.shape
    h = hash_coeffs.shape[0]
    t = bsz * seq

    ids_f = input_ids.reshape(-1)[None, :].astype(jnp.float32)
    idx = ((ids_f * hash_coeffs[:, None]) % MEMORY_SIZE).astype(jnp.int32)
    flat_idx = idx.reshape(-1)

    w16 = W.astype(jnp.bfloat16)
    b2 = b.reshape(1, d)
    hidden2d = hidden_states.reshape(t, d)

    multi = _sc_gather(memory_table, flat_idx)
    out = _tc_project(multi, w16, b2, hidden2d, gate)
    return out.reshape(bsz, seq, d)
